# baseline (speedup 1.0000x reference)
import functools

import jax
import jax.numpy as jnp
from jax.experimental import pallas as pl

_HI = jax.lax.Precision.HIGHEST
_F32 = jnp.float32


def _mm_kern(a_ref, w_ref, b_ref, o_ref, *, relu):
    acc = jnp.dot(a_ref[...], w_ref[...],
                  preferred_element_type=_F32) + b_ref[...]
    acc = jnp.maximum(acc, 0.0) if relu else acc
    o_ref[...] = acc.astype(o_ref.dtype)


def _mm(a, w, b, relu, bm, out_dtype=jnp.bfloat16):
    a = a.astype(jnp.bfloat16)
    w = w.astype(jnp.bfloat16)
    m, k = a.shape
    n = w.shape[1]
    assert m % bm == 0
    return pl.pallas_call(
        functools.partial(_mm_kern, relu=relu),
        grid=(m // bm,),
        in_specs=[
            pl.BlockSpec((bm, k), lambda i: (i, 0)),
            pl.BlockSpec((k, n), lambda i: (0, 0)),
            pl.BlockSpec((1, n), lambda i: (0, 0)),
        ],
        out_specs=pl.BlockSpec((bm, n), lambda i: (i, 0)),
        out_shape=jax.ShapeDtypeStruct((m, n), out_dtype),
    )(a, w, b.reshape(1, n))


def _vq_kern(h_ref, we3_ref, be3_ref, cb_ref, cbt_ref, wd1_ref, bd1_ref,
             g_ref, sse_ref, *, bm, ncodes):
    z = jnp.dot(h_ref[...], we3_ref[...],
                preferred_element_type=_F32) + be3_ref[...]
    zn = jnp.sum(z * z, axis=1, keepdims=True)
    cb = cb_ref[...]
    cn = jnp.sum(cb * cb, axis=1)[None, :]
    d = zn - 2.0 * jnp.dot(z.astype(jnp.bfloat16), cbt_ref[...],
                           preferred_element_type=_F32) + cn
    idx = jnp.argmin(d, axis=1)
    oh = (jax.lax.broadcasted_iota(jnp.int32, (bm, ncodes), 1)
          == idx[:, None]).astype(_F32)
    zq = jnp.dot(oh, cb, precision=_HI, preferred_element_type=_F32)
    se = jnp.sum((z - zq) ** 2)

    @pl.when(pl.program_id(0) == 0)
    def _():
        sse_ref[...] = jnp.zeros((1, 1), _F32)

    sse_ref[...] += se.reshape(1, 1)
    g = jnp.dot(zq.astype(jnp.bfloat16), wd1_ref[...],
                preferred_element_type=_F32) + bd1_ref[...]
    g_ref[...] = jnp.maximum(g, 0.0).astype(g_ref.dtype)


def _vq(h2, we3, be3, cb, wd1, bd1, bm):
    m, k = h2.shape
    ncodes, c = cb.shape
    n = wd1.shape[1]
    assert m % bm == 0
    g, sse = pl.pallas_call(
        functools.partial(_vq_kern, bm=bm, ncodes=ncodes),
        grid=(m // bm,),
        in_specs=[
            pl.BlockSpec((bm, k), lambda i: (i, 0)),
            pl.BlockSpec((k, c), lambda i: (0, 0)),
            pl.BlockSpec((1, c), lambda i: (0, 0)),
            pl.BlockSpec((ncodes, c), lambda i: (0, 0)),
            pl.BlockSpec((c, ncodes), lambda i: (0, 0)),
            pl.BlockSpec((c, n), lambda i: (0, 0)),
            pl.BlockSpec((1, n), lambda i: (0, 0)),
        ],
        out_specs=[
            pl.BlockSpec((bm, n), lambda i: (i, 0)),
            pl.BlockSpec((1, 1), lambda i: (0, 0)),
        ],
        out_shape=[
            jax.ShapeDtypeStruct((m, n), jnp.bfloat16),
            jax.ShapeDtypeStruct((1, 1), _F32),
        ],
    )(h2.astype(jnp.bfloat16), we3.astype(jnp.bfloat16), be3.reshape(1, c),
      cb, cb.T.astype(jnp.bfloat16), wd1.astype(jnp.bfloat16),
      bd1.reshape(1, n))
    return g, sse


def _conv3_kern(xf_ref, w_ref, b_ref, o_ref, *, wpad, ch, relu):
    base = pl.program_id(1) * ch
    cin = xf_ref.shape[-1]
    cout = o_ref.shape[-1]
    acc = jnp.zeros((ch, cout), _F32) + b_ref[...]
    for ky in range(3):
        for kx in range(3):
            off = ky * wpad + kx
            al, r = (off // 8) * 8, off % 8
            xs8 = xf_ref[0, pl.ds(base + al, ch + 8), :]
            xs = jax.lax.slice(xs8, (r, 0), (r + ch, cin))
            acc = acc + jnp.dot(xs, w_ref[ky * 3 + kx],
                                preferred_element_type=_F32)
    acc = jnp.maximum(acc, 0.0) if relu else acc
    o_ref[0] = acc.astype(o_ref.dtype)


def _conv3(ximg, w9, b, relu, nstrips, ch, out_dtype=jnp.bfloat16):
    bsz, hp, wp, cin = ximg.shape
    cout = w9.shape[2]
    hout = hp - 2
    assert hout % nstrips == 0
    rs = hout // nstrips
    ximg = ximg.astype(jnp.bfloat16)
    w9 = w9.astype(jnp.bfloat16)
    strips = jnp.stack([ximg[:, s * rs:s * rs + rs + 2] for s in range(nstrips)],
                       axis=1)
    nb = bsz * nstrips
    xf = jnp.pad(strips.reshape(nb, (rs + 2) * wp, cin), ((0, 0), (1, 15), (0, 0)))
    ln = (rs + 2) * wp + 16
    mrows = rs * wp
    assert mrows % ch == 0
    out = pl.pallas_call(
        functools.partial(_conv3_kern, wpad=wp, ch=ch, relu=relu),
        grid=(nb, mrows // ch),
        in_specs=[
            pl.BlockSpec((1, ln, cin), lambda i, j: (i, 0, 0)),
            pl.BlockSpec((9, cin, cout), lambda i, j: (0, 0, 0)),
            pl.BlockSpec((1, cout), lambda i, j: (0, 0)),
        ],
        out_specs=pl.BlockSpec((1, ch, cout), lambda i, j: (i, j, 0)),
        out_shape=jax.ShapeDtypeStruct((nb, mrows, cout), out_dtype),
    )(xf, w9, b.reshape(1, cout))
    out = out.reshape(bsz, hout, wp, cout)[:, :, 1:wp - 1, :]
    return out


def _patches4x4s2(xp, hout):
    ps = []
    for ky in range(4):
        for kx in range(4):
            ps.append(xp[:, ky:ky + 2 * hout - 1:2, kx:kx + 2 * hout - 1:2, :])
    return jnp.concatenate(ps, axis=-1)


def kernel(x, W_e1, b_e1, W_e2, b_e2, W_e3, b_e3, codebook,
           W_d1, b_d1, W_d2, b_d2, W_d3, b_d3):
    bsz = x.shape[0]
    xt = jnp.transpose(x, (0, 2, 3, 1))
    xp = jnp.pad(xt, ((0, 0), (1, 1), (1, 1), (0, 0)))
    p1 = _patches4x4s2(xp, 112).reshape(bsz * 112 * 112, 48)
    wf1 = W_e1.transpose(2, 3, 1, 0).reshape(48, 96)
    h1 = _mm(p1, wf1, b_e1, relu=True, bm=3584)
    h1i = h1.reshape(bsz, 112, 112, 96)
    h1p = jnp.pad(h1i, ((0, 0), (1, 1), (1, 1), (0, 0)))
    p2 = _patches4x4s2(h1p, 56).reshape(bsz * 56 * 56, 1536)
    wf2 = W_e2.transpose(2, 3, 1, 0).reshape(1536, 192)
    h2 = _mm(p2, wf2, b_e2, relu=True, bm=1792)
    we3 = W_e3.reshape(64, 192).T
    wd1 = W_d1.reshape(192, 64).T
    g1, sse = _vq(h2, we3, b_e3, codebook, wd1, b_d1, bm=1792)
    nz = h2.shape[0] * codebook.shape[1]
    loss = (sse[0, 0] * jnp.float32(1.25 / nz)).reshape(())
    g1i = g1.reshape(bsz, 56, 56, 192)
    u1 = jnp.repeat(jnp.repeat(g1i, 2, axis=1), 2, axis=2)
    u1p = jnp.pad(u1, ((0, 0), (1, 1), (1, 1), (0, 0)))
    w9d2 = W_d2.transpose(2, 3, 1, 0).reshape(9, 192, 96)
    g2 = _conv3(u1p, w9d2, b_d2, relu=True, nstrips=4, ch=3192)
    u2 = jnp.repeat(jnp.repeat(g2, 2, axis=1), 2, axis=2)
    u2p = jnp.pad(u2, ((0, 0), (1, 1), (1, 1), (0, 0)))
    w9d3 = W_d3.transpose(2, 3, 1, 0).reshape(9, 96, 3)
    xh = _conv3(u2p, w9d3, b_d3, relu=False, nstrips=4, ch=1808,
                out_dtype=_F32)
    x_hat = jnp.transpose(xh, (0, 3, 1, 2))
    return (x_hat, loss)


# speedup vs baseline: 1.2539x; 1.2539x over previous
import functools

import jax
import jax.numpy as jnp
from jax.experimental import pallas as pl

_HI = jax.lax.Precision.HIGHEST
_F32 = jnp.float32


def _mm_kern(a_ref, w_ref, b_ref, o_ref, *, relu):
    acc = jnp.dot(a_ref[...], w_ref[...],
                  preferred_element_type=_F32) + b_ref[...]
    acc = jnp.maximum(acc, 0.0) if relu else acc
    o_ref[...] = acc.astype(o_ref.dtype)


def _mm(a, w, b, relu, bm, out_dtype=jnp.bfloat16):
    a = a.astype(jnp.bfloat16)
    w = w.astype(jnp.bfloat16)
    m, k = a.shape
    n = w.shape[1]
    assert m % bm == 0
    return pl.pallas_call(
        functools.partial(_mm_kern, relu=relu),
        grid=(m // bm,),
        in_specs=[
            pl.BlockSpec((bm, k), lambda i: (i, 0)),
            pl.BlockSpec((k, n), lambda i: (0, 0)),
            pl.BlockSpec((1, n), lambda i: (0, 0)),
        ],
        out_specs=pl.BlockSpec((bm, n), lambda i: (i, 0)),
        out_shape=jax.ShapeDtypeStruct((m, n), out_dtype),
    )(a, w, b.reshape(1, n))


def _vq_kern(h_ref, we3_ref, be3_ref, cb_ref, cbt_ref, wd1_ref, bd1_ref,
             g_ref, sse_ref, *, bm, ncodes):
    z = jnp.dot(h_ref[...], we3_ref[...],
                preferred_element_type=_F32) + be3_ref[...]
    zn = jnp.sum(z * z, axis=1, keepdims=True)
    cb = cb_ref[...]
    cn = jnp.sum(cb * cb, axis=1)[None, :]
    d = zn - 2.0 * jnp.dot(z.astype(jnp.bfloat16), cbt_ref[...],
                           preferred_element_type=_F32) + cn
    idx = jnp.argmin(d, axis=1)
    oh = (jax.lax.broadcasted_iota(jnp.int32, (bm, ncodes), 1)
          == idx[:, None]).astype(_F32)
    zq = jnp.dot(oh, cb, precision=_HI, preferred_element_type=_F32)
    se = jnp.sum((z - zq) ** 2)

    @pl.when(pl.program_id(0) == 0)
    def _():
        sse_ref[...] = jnp.zeros((1, 1), _F32)

    sse_ref[...] += se.reshape(1, 1)
    g = jnp.dot(zq.astype(jnp.bfloat16), wd1_ref[...],
                preferred_element_type=_F32) + bd1_ref[...]
    g_ref[...] = jnp.maximum(g, 0.0).astype(g_ref.dtype)


def _vq(h2, we3, be3, cb, wd1, bd1, bm):
    m, k = h2.shape
    ncodes, c = cb.shape
    n = wd1.shape[1]
    assert m % bm == 0
    g, sse = pl.pallas_call(
        functools.partial(_vq_kern, bm=bm, ncodes=ncodes),
        grid=(m // bm,),
        in_specs=[
            pl.BlockSpec((bm, k), lambda i: (i, 0)),
            pl.BlockSpec((k, c), lambda i: (0, 0)),
            pl.BlockSpec((1, c), lambda i: (0, 0)),
            pl.BlockSpec((ncodes, c), lambda i: (0, 0)),
            pl.BlockSpec((c, ncodes), lambda i: (0, 0)),
            pl.BlockSpec((c, n), lambda i: (0, 0)),
            pl.BlockSpec((1, n), lambda i: (0, 0)),
        ],
        out_specs=[
            pl.BlockSpec((bm, n), lambda i: (i, 0)),
            pl.BlockSpec((1, 1), lambda i: (0, 0)),
        ],
        out_shape=[
            jax.ShapeDtypeStruct((m, n), jnp.bfloat16),
            jax.ShapeDtypeStruct((1, 1), _F32),
        ],
    )(h2.astype(jnp.bfloat16), we3.astype(jnp.bfloat16), be3.reshape(1, c),
      cb, cb.T.astype(jnp.bfloat16), wd1.astype(jnp.bfloat16),
      bd1.reshape(1, n))
    return g, sse


def _d2_kern(xf_ref, w_ref, b_ref, o_ref, *, wpad, ch):
    base = pl.program_id(1) * ch
    cin = xf_ref.shape[-1]
    nout = o_ref.shape[-1]
    acc = jnp.zeros((ch, nout), _F32) + b_ref[...]
    for dy in range(2):
        for dx in range(2):
            off = dy * wpad + dx
            al, r = (off // 8) * 8, off % 8
            xs8 = xf_ref[0, pl.ds(base + al, ch + 8), :]
            xs = jax.lax.slice(xs8, (r, 0), (r + ch, cin))
            acc = acc + jnp.dot(xs, w_ref[dy * 2 + dx],
                                preferred_element_type=_F32)
    o_ref[0] = jnp.maximum(acc, 0.0).astype(o_ref.dtype)


def _d2(xf, w4, b4, wpad, mrows, ch):
    nb, ln, cin = xf.shape
    nout = w4.shape[2]
    return pl.pallas_call(
        functools.partial(_d2_kern, wpad=wpad, ch=ch),
        grid=(nb, mrows // ch),
        in_specs=[
            pl.BlockSpec((1, ln, cin), lambda i, j: (i, 0, 0)),
            pl.BlockSpec((4, cin, nout), lambda i, j: (0, 0, 0)),
            pl.BlockSpec((1, nout), lambda i, j: (0, 0)),
        ],
        out_specs=pl.BlockSpec((1, ch, nout), lambda i, j: (i, j, 0)),
        out_shape=jax.ShapeDtypeStruct((nb, mrows, nout), jnp.bfloat16),
    )(xf.astype(jnp.bfloat16), w4.astype(jnp.bfloat16), b4)


def _d3_kern(xf_ref, w_ref, b_ref, o_ref, *, wpad, ch, cout):
    base = pl.program_id(1) * ch
    xs = xf_ref[0, pl.ds(base, ch + 120), :]
    y = jnp.dot(xs, w_ref[...], preferred_element_type=_F32)
    npar = 4 * cout
    acc = jnp.zeros((ch, npar), _F32) + b_ref[...]
    for k, off in enumerate((0, 1, wpad, wpad + 1)):
        acc = acc + jax.lax.slice(y, (off, npar * k), (off + ch, npar * (k + 1)))
    o_ref[0] = acc


def _d3(xf, w48, b12, wpad, mrows, ch, cout):
    nb, ln, cin = xf.shape
    npar = 4 * cout
    return pl.pallas_call(
        functools.partial(_d3_kern, wpad=wpad, ch=ch, cout=cout),
        grid=(nb, mrows // ch),
        in_specs=[
            pl.BlockSpec((1, ln, cin), lambda i, j: (i, 0, 0)),
            pl.BlockSpec((cin, 4 * npar), lambda i, j: (0, 0)),
            pl.BlockSpec((1, npar), lambda i, j: (0, 0)),
        ],
        out_specs=pl.BlockSpec((1, ch, npar), lambda i, j: (i, j, 0)),
        out_shape=jax.ShapeDtypeStruct((nb, mrows, npar), _F32),
    )(xf.astype(jnp.bfloat16), w48.astype(jnp.bfloat16), b12)


_SEL = None


def _fold_up2(w):
    sel = jnp.array([[[1, 0, 0], [0, 1, 1]],
                     [[1, 1, 0], [0, 0, 1]]], _F32)
    return jnp.einsum('ajy,bkx,ocyx->jkcabo', sel, sel, w, precision=_HI)


def _flatpad(img):
    b, hp, wp, c = img.shape
    return jnp.pad(img.reshape(b, hp * wp, c), ((0, 0), (0, 16), (0, 0)))


def _deinterleave(y, hh, wpad, cout, hout):
    b = y.shape[0]
    yr = y[:, :hh * wpad + wpad].reshape(b, hh + 1, wpad, 2, 2, cout)
    parts = [yr[:, a:a + hh, bb:bb + hh, a, bb, :]
             for a in range(2) for bb in range(2)]
    st = jnp.stack(parts, axis=3).reshape(b, hh, hh, 2, 2, cout)
    return st.transpose(0, 1, 3, 2, 4, 5).reshape(b, 2 * hh, 2 * hh, cout)


def _patches4x4s2(xp, hout):
    ps = []
    for ky in range(4):
        for kx in range(4):
            ps.append(xp[:, ky:ky + 2 * hout - 1:2, kx:kx + 2 * hout - 1:2, :])
    return jnp.concatenate(ps, axis=-1)


def kernel(x, W_e1, b_e1, W_e2, b_e2, W_e3, b_e3, codebook,
           W_d1, b_d1, W_d2, b_d2, W_d3, b_d3):
    bsz = x.shape[0]
    xt = jnp.transpose(x, (0, 2, 3, 1))
    xp = jnp.pad(xt, ((0, 0), (1, 1), (1, 1), (0, 0)))
    p1 = _patches4x4s2(xp, 112).reshape(bsz * 112 * 112, 48)
    wf1 = W_e1.transpose(2, 3, 1, 0).reshape(48, 96)
    h1 = _mm(p1, wf1, b_e1, relu=True, bm=3584)
    h1i = h1.reshape(bsz, 112, 112, 96)
    h1p = jnp.pad(h1i, ((0, 0), (1, 1), (1, 1), (0, 0)))
    p2 = _patches4x4s2(h1p, 56).reshape(bsz * 56 * 56, 1536)
    wf2 = W_e2.transpose(2, 3, 1, 0).reshape(1536, 192)
    h2 = _mm(p2, wf2, b_e2, relu=True, bm=1792)
    we3 = W_e3.reshape(64, 192).T
    wd1 = W_d1.reshape(192, 64).T
    g1, sse = _vq(h2, we3, b_e3, codebook, wd1, b_d1, bm=1792)
    nz = h2.shape[0] * codebook.shape[1]
    loss = (sse[0, 0] * jnp.float32(1.25 / nz)).reshape(())
    g1p = jnp.pad(g1.reshape(bsz, 56, 56, 192), ((0, 0), (1, 1), (1, 1), (0, 0)))
    w4d2 = _fold_up2(W_d2).reshape(4, 192, 384)
    b384 = jnp.tile(b_d2, 4).reshape(1, 384)
    y2 = _d2(_flatpad(g1p), w4d2, b384, wpad=58, mrows=3312, ch=1104)
    g2 = _deinterleave(y2, 56, 58, 96, 112)
    g2p = jnp.pad(g2, ((0, 0), (1, 1), (1, 1), (0, 0)))
    w48 = _fold_up2(W_d3).transpose(2, 0, 1, 3, 4, 5).reshape(96, 48)
    b12 = jnp.tile(b_d3, 4).reshape(1, 12)
    y3 = _d3(_flatpad(g2p), w48, b12, wpad=114, mrows=12888, ch=4296, cout=3)
    xh = _deinterleave(y3, 112, 114, 3, 224)
    x_hat = jnp.transpose(xh, (0, 3, 1, 2))
    return (x_hat, loss)


# speedup vs baseline: 3.7746x; 3.0103x over previous
import functools

import jax
import jax.numpy as jnp
from jax.experimental import pallas as pl

_HI = jax.lax.Precision.HIGHEST
_F32 = jnp.float32


def _mm_kern(a_ref, w_ref, b_ref, o_ref, *, relu):
    acc = jnp.dot(a_ref[...], w_ref[...],
                  preferred_element_type=_F32) + b_ref[...]
    acc = jnp.maximum(acc, 0.0) if relu else acc
    o_ref[...] = acc.astype(o_ref.dtype)


def _mm(a, w, b, relu, bm, out_dtype=jnp.bfloat16):
    a = a.astype(jnp.bfloat16)
    w = w.astype(jnp.bfloat16)
    m, k = a.shape
    n = w.shape[1]
    assert m % bm == 0
    return pl.pallas_call(
        functools.partial(_mm_kern, relu=relu),
        grid=(m // bm,),
        in_specs=[
            pl.BlockSpec((bm, k), lambda i: (i, 0)),
            pl.BlockSpec((k, n), lambda i: (0, 0)),
            pl.BlockSpec((1, n), lambda i: (0, 0)),
        ],
        out_specs=pl.BlockSpec((bm, n), lambda i: (i, 0)),
        out_shape=jax.ShapeDtypeStruct((m, n), out_dtype),
    )(a, w, b.reshape(1, n))


def _vq_kern(h_ref, we3_ref, be3_ref, cb_ref, cbt_ref, cbb_ref, wd1_ref,
             bd1_ref, g_ref, sse_ref, *, bm, ncodes):
    z = jnp.dot(h_ref[...], we3_ref[...],
                preferred_element_type=_F32) + be3_ref[...]
    zn = jnp.sum(z * z, axis=1, keepdims=True)
    cb = cb_ref[...]
    cn = jnp.sum(cb * cb, axis=1)[None, :]
    d = zn - 2.0 * jnp.dot(z.astype(jnp.bfloat16), cbt_ref[...],
                           preferred_element_type=_F32) + cn
    idx = jnp.argmin(d, axis=1)
    oh = (jax.lax.broadcasted_iota(jnp.int32, (bm, ncodes), 1)
          == idx[:, None]).astype(_F32)
    zq = jnp.dot(oh.astype(jnp.bfloat16), cbb_ref[...],
                 preferred_element_type=_F32)
    se = jnp.sum((z - zq) ** 2)

    @pl.when(pl.program_id(0) == 0)
    def _():
        sse_ref[...] = jnp.zeros((1, 1), _F32)

    sse_ref[...] += se.reshape(1, 1)
    g = jnp.dot(zq.astype(jnp.bfloat16), wd1_ref[...],
                preferred_element_type=_F32) + bd1_ref[...]
    g_ref[...] = jnp.maximum(g, 0.0).astype(g_ref.dtype)


def _vq(h2, we3, be3, cb, wd1, bd1, bm):
    m, k = h2.shape
    ncodes, c = cb.shape
    n = wd1.shape[1]
    assert m % bm == 0
    g, sse = pl.pallas_call(
        functools.partial(_vq_kern, bm=bm, ncodes=ncodes),
        grid=(m // bm,),
        in_specs=[
            pl.BlockSpec((bm, k), lambda i: (i, 0)),
            pl.BlockSpec((k, c), lambda i: (0, 0)),
            pl.BlockSpec((1, c), lambda i: (0, 0)),
            pl.BlockSpec((ncodes, c), lambda i: (0, 0)),
            pl.BlockSpec((c, ncodes), lambda i: (0, 0)),
            pl.BlockSpec((ncodes, c), lambda i: (0, 0)),
            pl.BlockSpec((c, n), lambda i: (0, 0)),
            pl.BlockSpec((1, n), lambda i: (0, 0)),
        ],
        out_specs=[
            pl.BlockSpec((bm, n), lambda i: (i, 0)),
            pl.BlockSpec((1, 1), lambda i: (0, 0)),
        ],
        out_shape=[
            jax.ShapeDtypeStruct((m, n), jnp.bfloat16),
            jax.ShapeDtypeStruct((1, 1), _F32),
        ],
    )(h2.astype(jnp.bfloat16), we3.astype(jnp.bfloat16), be3.reshape(1, c),
      cb, cb.T.astype(jnp.bfloat16), cb.astype(jnp.bfloat16),
      wd1.astype(jnp.bfloat16), bd1.reshape(1, n))
    return g, sse


def _d2_kern(xf_ref, w_ref, b_ref, o_ref, *, wpad, ch):
    base = pl.program_id(1) * ch
    cin = xf_ref.shape[-1]
    nout = o_ref.shape[-1]
    acc = jnp.zeros((ch, nout), _F32) + b_ref[...]
    for dy in range(2):
        for dx in range(2):
            off = dy * wpad + dx
            al, r = (off // 8) * 8, off % 8
            xs8 = xf_ref[0, pl.ds(base + al, ch + 8), :]
            xs = jax.lax.slice(xs8, (r, 0), (r + ch, cin))
            acc = acc + jnp.dot(xs, w_ref[dy * 2 + dx],
                                preferred_element_type=_F32)
    o_ref[0] = jnp.maximum(acc, 0.0).astype(o_ref.dtype)


def _d2(xf, w4, b4, wpad, mrows, ch):
    nb, ln, cin = xf.shape
    nout = w4.shape[2]
    return pl.pallas_call(
        functools.partial(_d2_kern, wpad=wpad, ch=ch),
        grid=(nb, mrows // ch),
        in_specs=[
            pl.BlockSpec((1, ln, cin), lambda i, j: (i, 0, 0)),
            pl.BlockSpec((4, cin, nout), lambda i, j: (0, 0, 0)),
            pl.BlockSpec((1, nout), lambda i, j: (0, 0)),
        ],
        out_specs=pl.BlockSpec((1, ch, nout), lambda i, j: (i, j, 0)),
        out_shape=jax.ShapeDtypeStruct((nb, mrows, nout), jnp.bfloat16),
    )(xf.astype(jnp.bfloat16), w4.astype(jnp.bfloat16), b4)


def _d3_kern(xf_ref, w_ref, b_ref, o_ref, *, wpad, ch, cout):
    base = pl.program_id(1) * ch
    xs = xf_ref[0, pl.ds(base, ch + 120), :]
    y = jnp.dot(xs, w_ref[...], preferred_element_type=_F32)
    npar = 4 * cout
    acc = jnp.zeros((ch, npar), _F32) + b_ref[...]
    for k, off in enumerate((0, 1, wpad, wpad + 1)):
        acc = acc + jax.lax.slice(y, (off, npar * k), (off + ch, npar * (k + 1)))
    o_ref[0] = acc


def _d3(xf, w48, b12, wpad, mrows, ch, cout):
    nb, ln, cin = xf.shape
    npar = 4 * cout
    return pl.pallas_call(
        functools.partial(_d3_kern, wpad=wpad, ch=ch, cout=cout),
        grid=(nb, mrows // ch),
        in_specs=[
            pl.BlockSpec((1, ln, cin), lambda i, j: (i, 0, 0)),
            pl.BlockSpec((cin, 4 * npar), lambda i, j: (0, 0)),
            pl.BlockSpec((1, npar), lambda i, j: (0, 0)),
        ],
        out_specs=pl.BlockSpec((1, ch, npar), lambda i, j: (i, j, 0)),
        out_shape=jax.ShapeDtypeStruct((nb, mrows, npar), _F32),
    )(xf.astype(jnp.bfloat16), w48.astype(jnp.bfloat16), b12)


_SEL = None


def _fold_up2(w):
    sel = jnp.array([[[1, 0, 0], [0, 1, 1]],
                     [[1, 1, 0], [0, 0, 1]]], _F32)
    return jnp.einsum('ajy,bkx,ocyx->jkcabo', sel, sel, w, precision=_HI)


def _flatpad(img):
    b, hp, wp, c = img.shape
    return jnp.pad(img.reshape(b, hp * wp, c), ((0, 0), (0, 16), (0, 0)))


def _deinterleave(y, hh, wpad, cout, hout):
    b = y.shape[0]
    yr = y[:, :hh * wpad + wpad].reshape(b, hh + 1, wpad, 2, 2, cout)
    parts = [yr[:, a:a + hh, bb:bb + hh, a, bb, :]
             for a in range(2) for bb in range(2)]
    st = jnp.stack(parts, axis=3).reshape(b, hh, hh, 2, 2, cout)
    return st.transpose(0, 1, 3, 2, 4, 5).reshape(b, 2 * hh, 2 * hh, cout)


def _patches4x4s2(xp, hout):
    ps = []
    for ky in range(4):
        for kx in range(4):
            ps.append(xp[:, ky:ky + 2 * hout - 1:2, kx:kx + 2 * hout - 1:2, :])
    return jnp.concatenate(ps, axis=-1)


def kernel(x, W_e1, b_e1, W_e2, b_e2, W_e3, b_e3, codebook,
           W_d1, b_d1, W_d2, b_d2, W_d3, b_d3):
    bsz = x.shape[0]
    xt = jnp.transpose(x, (0, 2, 3, 1))
    xp = jnp.pad(xt, ((0, 0), (1, 1), (1, 1), (0, 0))).astype(jnp.bfloat16)
    p1 = _patches4x4s2(xp, 112).reshape(bsz * 112 * 112, 48)
    wf1 = W_e1.transpose(2, 3, 1, 0).reshape(48, 96)
    h1 = _mm(p1, wf1, b_e1, relu=True, bm=3584)
    h1p = jnp.pad(h1.reshape(bsz, 112, 112, 96),
                  ((0, 0), (1, 1), (1, 1), (0, 0)))
    rl = h1p.reshape(bsz, 57, 2, 57, 2, 96).transpose(0, 1, 3, 2, 4, 5)
    rlf = jnp.pad(rl.reshape(bsz, 3249, 384), ((0, 0), (0, 16), (0, 0)))
    w4e2 = (W_e2.transpose(2, 3, 1, 0).reshape(2, 2, 2, 2, 96, 192)
            .transpose(0, 2, 1, 3, 4, 5).reshape(4, 384, 192))
    y2e = _d2(rlf, w4e2, b_e2.reshape(1, 192), wpad=57, mrows=3192, ch=1064)
    h2 = y2e.reshape(bsz, 56, 57, 192)[:, :, :56, :].reshape(bsz * 3136, 192)
    we3 = W_e3.reshape(64, 192).T
    wd1 = W_d1.reshape(192, 64).T
    g1, sse = _vq(h2, we3, b_e3, codebook, wd1, b_d1, bm=1792)
    nz = h2.shape[0] * codebook.shape[1]
    loss = (sse[0, 0] * jnp.float32(1.25 / nz)).reshape(())
    g1p = jnp.pad(g1.reshape(bsz, 56, 56, 192), ((0, 0), (1, 1), (1, 1), (0, 0)))
    w4d2 = _fold_up2(W_d2).reshape(4, 192, 384)
    b384 = jnp.tile(b_d2, 4).reshape(1, 384)
    y2 = _d2(_flatpad(g1p), w4d2, b384, wpad=58, mrows=3312, ch=1104)
    g2 = _deinterleave(y2, 56, 58, 96, 112)
    g2p = jnp.pad(g2, ((0, 0), (1, 1), (1, 1), (0, 0)))
    w48 = _fold_up2(W_d3).transpose(2, 0, 1, 3, 4, 5).reshape(96, 48)
    b12 = jnp.tile(b_d3, 4).reshape(1, 12)
    y3 = _d3(_flatpad(g2p), w48, b12, wpad=114, mrows=12888, ch=4296, cout=3)
    xh = _deinterleave(y3, 112, 114, 3, 224)
    x_hat = jnp.transpose(xh, (0, 3, 1, 2))
    return (x_hat, loss)
